# Initial kernel scaffold; baseline (speedup 1.0000x reference)
import jax
import jax.numpy as jnp
from jax import lax
from jax.experimental import pallas as pl
from jax.experimental.pallas import tpu as pltpu
from jax.experimental.pallas import tpu_sc as plsc

B, N, D, H, W, C = 4, 6, 41, 14, 30, 64
NXG, NYG = 200, 200
NP = B * N * D * H * W
PPB = NP // B
CHUNK = 1024
NFULL = PPB // CHUNK
TAIL = PPB - NFULL * CHUNK
NTILES = 16
GROWS = 40016
DUMP = 40000
CH = 32


def _voxelize(fx, fy, fz):
    ix = ((fx + 50.0) / 0.5).astype(jnp.int32)
    iy = ((fy + 50.0) / 0.5).astype(jnp.int32)
    iz = ((fz + 10.0) / 20.0).astype(jnp.int32)
    kept = ((ix >= 0) & (ix < NXG) & (iy >= 0) & (iy < NYG) & (iz == 0))
    return jnp.where(kept, ix * NYG + iy, DUMP)


def _sc_body(x_ref, g_ref, out_ref, grid, featb, zbuf, gbuf, idx2d,
             idx_tail, sem):
    cid = lax.axis_index("c")
    sid = lax.axis_index("s")
    c0 = cid * CH
    lane = lax.iota(jnp.int32, 16)

    def _zb(i, _):
        zbuf[i, pl.ds(0, 16)] = jnp.zeros((16,), jnp.float32)
        zbuf[i, pl.ds(16, 16)] = jnp.zeros((16,), jnp.float32)
        return 0
    lax.fori_loop(0, CHUNK, _zb, 0)

    def _geom_vec(i):
        off = i * 48
        fx = plsc.load_gather(gbuf, [lane * 3 + off])
        fy = plsc.load_gather(gbuf, [lane * 3 + (off + 1)])
        fz = plsc.load_gather(gbuf, [lane * 3 + (off + 2)])
        return _voxelize(fx, fy, fz)

    def _do_chunk(base, npts):
        cp = pltpu.async_copy(
            x_ref.at[pl.ds(base, npts), pl.ds(c0, CH)],
            featb.at[pl.ds(0, npts)], sem)
        pltpu.sync_copy(g_ref.at[pl.ds(base * 3, npts * 3)],
                        gbuf.at[pl.ds(0, npts * 3)])
        nvec_full = (npts // 128) * 8

        def _vec(i, _):
            idx2d[i // 8, pl.ds((i % 8) * 16, 16)] = _geom_vec(i)
            return 0
        lax.fori_loop(0, nvec_full, _vec, 0)

        if npts % 128:
            rem = npts - (npts // 128) * 128
            v0 = nvec_full
            idx_tail[pl.ds(0, 16)] = _geom_vec(v0)
            val = _geom_vec(v0 + 1)
            plsc.store_scatter(idx_tail, [lane + 16], val,
                               mask=lane < rem - 16)
        cp.wait()
        for j in range(npts // 128):
            pltpu.sync_copy(featb.at[pl.ds(j * 128, 128)],
                            grid.at[idx2d.at[j]], add=True)
        if npts % 128:
            pltpu.sync_copy(featb.at[pl.ds((npts // 128) * 128, rem)],
                            grid.at[idx_tail], add=True)

    for b in range(B):
        r0 = sid * 2501
        pltpu.sync_copy(zbuf, grid.at[pl.ds(r0, CHUNK)])
        pltpu.sync_copy(zbuf, grid.at[pl.ds(r0 + CHUNK, CHUNK)])
        pltpu.sync_copy(zbuf.at[pl.ds(0, 2501 - 2 * CHUNK)],
                        grid.at[pl.ds(r0 + 2 * CHUNK, 2501 - 2 * CHUNK)])
        plsc.subcore_barrier()

        bb = b * PPB

        def _chunk_loop(t, _):
            q = t * NTILES + sid

            @pl.when(q < NFULL)
            def _():
                _do_chunk(bb + q * CHUNK, CHUNK)
            return 0
        lax.fori_loop(0, (NFULL + NTILES - 1) // NTILES, _chunk_loop, 0)

        @pl.when(sid == NTILES - 1)
        def _():
            _do_chunk(bb + NFULL * CHUNK, TAIL)
        plsc.subcore_barrier()

        o0 = sid * 2500
        pltpu.sync_copy(grid.at[pl.ds(o0, 2500)],
                        out_ref.at[b, pl.ds(o0, 2500), pl.ds(c0, CH)])
        plsc.subcore_barrier()


_sc_call = pl.kernel(
    _sc_body,
    out_type=jax.ShapeDtypeStruct((B, NXG * NYG, C), jnp.float32),
    mesh=plsc.VectorSubcoreMesh(core_axis_name="c", subcore_axis_name="s"),
    scratch_types=[
        pltpu.VMEM_SHARED((GROWS, CH), jnp.float32),
        pltpu.VMEM((CHUNK, CH), jnp.float32),
        pltpu.VMEM((CHUNK, CH), jnp.float32),
        pltpu.VMEM((3 * CHUNK,), jnp.float32),
        pltpu.VMEM((8, 128), jnp.int32),
        pltpu.VMEM((24,), jnp.int32),
        pltpu.SemaphoreType.DMA,
    ],
)


def _tr_body(v_ref, o_ref):
    o_ref[...] = jnp.swapaxes(v_ref[...], 1, 2)


_tr_call = pl.pallas_call(
    _tr_body,
    grid=(B, 20),
    in_specs=[pl.BlockSpec((1, 2000, C), lambda b, j: (b, j, 0))],
    out_specs=pl.BlockSpec((1, C, 2000), lambda b, j: (b, 0, j)),
    out_shape=jax.ShapeDtypeStruct((B, C, NXG * NYG), jnp.float32),
)


def kernel(x, geom_feats):
    xf = x.reshape(NP, C)
    gf = geom_feats.reshape(NP * 3)
    vox = _sc_call(xf, gf)
    out = _tr_call(vox)
    return out.reshape(B, C, NXG, NYG)


# speedup vs baseline: 1.0987x; 1.0987x over previous
import jax
import jax.numpy as jnp
from jax import lax
from jax.experimental import pallas as pl
from jax.experimental.pallas import tpu as pltpu
from jax.experimental.pallas import tpu_sc as plsc

B, N, D, H, W, C = 4, 6, 41, 14, 30, 64
NXG, NYG = 200, 200
NP = B * N * D * H * W
PPB = NP // B
CHUNK = 1024
NFULL = PPB // CHUNK
TAIL = PPB - NFULL * CHUNK
NTILES = 16
GROWS = 40016
DUMP = 40000
CH = 32


def _voxelize(fx, fy, fz):
    ix = ((fx + 50.0) / 0.5).astype(jnp.int32)
    iy = ((fy + 50.0) / 0.5).astype(jnp.int32)
    iz = ((fz + 10.0) / 20.0).astype(jnp.int32)
    kept = ((ix >= 0) & (ix < NXG) & (iy >= 0) & (iy < NYG) & (iz == 0))
    return jnp.where(kept, ix * NYG + iy, DUMP)


def _sc_body(x_ref, g_ref, out_ref, grid, featb, gbuf, idx2d,
             idx_tail, sem):
    cid = lax.axis_index("c")
    sid = lax.axis_index("s")
    c0 = cid * CH
    lane = lax.iota(jnp.int32, 16)

    def _geom_vec(i):
        off = i * 48
        fx = plsc.load_gather(gbuf, [lane * 3 + off])
        fy = plsc.load_gather(gbuf, [lane * 3 + (off + 1)])
        fz = plsc.load_gather(gbuf, [lane * 3 + (off + 2)])
        return _voxelize(fx, fy, fz)

    def _do_chunk(base, npts):
        cp = pltpu.async_copy(
            x_ref.at[pl.ds(base, npts), pl.ds(c0, CH)],
            featb.at[pl.ds(0, npts)], sem)
        pltpu.sync_copy(g_ref.at[pl.ds(base * 3, npts * 3)],
                        gbuf.at[pl.ds(0, npts * 3)])
        nvec_full = (npts // 128) * 8

        def _vec(i, _):
            idx2d[i // 8, pl.ds((i % 8) * 16, 16)] = _geom_vec(i)
            return 0
        lax.fori_loop(0, nvec_full, _vec, 0)

        if npts % 128:
            rem = npts - (npts // 128) * 128
            v0 = nvec_full
            idx_tail[pl.ds(0, 16)] = _geom_vec(v0)
            val = _geom_vec(v0 + 1)
            plsc.store_scatter(idx_tail, [lane + 16], val,
                               mask=lane < rem - 16)
        cp.wait()
        for j in range(npts // 128):
            pltpu.sync_copy(featb.at[pl.ds(j * 128, 128)],
                            grid.at[idx2d.at[j]], add=True)
        if npts % 128:
            pltpu.sync_copy(featb.at[pl.ds((npts // 128) * 128, rem)],
                            grid.at[idx_tail], add=True)

    for b in range(B):
        def _zb(i, _):
            featb[i, pl.ds(0, 16)] = jnp.zeros((16,), jnp.float32)
            featb[i, pl.ds(16, 16)] = jnp.zeros((16,), jnp.float32)
            return 0
        lax.fori_loop(0, CHUNK, _zb, 0)
        r0 = sid * 2501
        pltpu.sync_copy(featb, grid.at[pl.ds(r0, CHUNK)])
        pltpu.sync_copy(featb, grid.at[pl.ds(r0 + CHUNK, CHUNK)])
        pltpu.sync_copy(featb.at[pl.ds(0, 2501 - 2 * CHUNK)],
                        grid.at[pl.ds(r0 + 2 * CHUNK, 2501 - 2 * CHUNK)])
        plsc.subcore_barrier()

        bb = b * PPB

        def _chunk_loop(t, _):
            q = t * NTILES + sid

            @pl.when(q < NFULL)
            def _():
                _do_chunk(bb + q * CHUNK, CHUNK)
            return 0
        lax.fori_loop(0, (NFULL + NTILES - 1) // NTILES, _chunk_loop, 0)

        @pl.when(sid == NTILES - 1)
        def _():
            _do_chunk(bb + NFULL * CHUNK, TAIL)
        plsc.subcore_barrier()

        o0 = sid * 2500
        pltpu.sync_copy(grid.at[pl.ds(o0, 2500)],
                        out_ref.at[b, pl.ds(o0, 2500), pl.ds(c0, CH)])
        plsc.subcore_barrier()


_sc_call = pl.kernel(
    _sc_body,
    out_type=jax.ShapeDtypeStruct((B, NXG * NYG, C), jnp.float32),
    mesh=plsc.VectorSubcoreMesh(core_axis_name="c", subcore_axis_name="s"),
    scratch_types=[
        pltpu.VMEM_SHARED((GROWS, CH), jnp.float32),
        pltpu.VMEM((CHUNK, CH), jnp.float32),
        pltpu.VMEM((3 * CHUNK,), jnp.float32),
        pltpu.VMEM((8, 128), jnp.int32),
        pltpu.VMEM((24,), jnp.int32),
        pltpu.SemaphoreType.DMA,
    ],
    compiler_params=pltpu.CompilerParams(use_tc_tiling_on_sc=False,
                                         needs_layout_passes=False),
)


_NJ = (NXG * NYG) // 128
_REM = NXG * NYG - _NJ * 128


def _tr_body(v_ref, o_ref, tbuf, tbuf2, sem):
    b = pl.program_id(0)
    j = pl.program_id(1)

    @pl.when(j < _NJ)
    def _():
        tbuf[...] = v_ref[0].T
        cp = pltpu.make_async_copy(
            tbuf, o_ref.at[b, :, pl.ds(j * 128, 128)], sem)
        cp.start()
        cp.wait()

    @pl.when(j == _NJ)
    def _():
        tbuf2[...] = v_ref[0, :_REM, :].T
        cp = pltpu.make_async_copy(
            tbuf2, o_ref.at[b, :, pl.ds(_NJ * 128, _REM)], sem)
        cp.start()
        cp.wait()


_tr_call = pl.pallas_call(
    _tr_body,
    grid=(B, _NJ + 1),
    in_specs=[pl.BlockSpec((1, 128, C), lambda b, j: (b, j, 0))],
    out_specs=pl.BlockSpec(memory_space=pltpu.MemorySpace.HBM),
    out_shape=jax.ShapeDtypeStruct((B, C, NXG * NYG), jnp.float32),
    scratch_shapes=[pltpu.VMEM((C, 128), jnp.float32),
                    pltpu.VMEM((C, _REM), jnp.float32),
                    pltpu.SemaphoreType.DMA],
)


def kernel(x, geom_feats):
    xf = x.reshape(NP, C)
    gf = geom_feats.reshape(NP * 3)
    vox = _sc_call(xf, gf)
    out = _tr_call(vox)
    return out.reshape(B, C, NXG, NYG)


# speedup vs baseline: 2.1340x; 1.9422x over previous
import jax
import jax.numpy as jnp
from jax import lax
from jax.experimental import pallas as pl
from jax.experimental.pallas import tpu as pltpu
from jax.experimental.pallas import tpu_sc as plsc

B, N, D, H, W, C = 4, 6, 41, 14, 30, 64
NXG, NYG = 200, 200
NP = B * N * D * H * W
PPB = NP // B
CHUNK = 1024
NFULL = PPB // CHUNK
TAIL = PPB - NFULL * CHUNK
NTILES = 16
GROWS = 40016
DUMP = 40000
CH = 32


def _voxelize(fx, fy, fz):
    ix = ((fx + 50.0) / 0.5).astype(jnp.int32)
    iy = ((fy + 50.0) / 0.5).astype(jnp.int32)
    iz = ((fz + 10.0) / 20.0).astype(jnp.int32)
    kept = ((ix >= 0) & (ix < NXG) & (iy >= 0) & (iy < NYG) & (iz == 0))
    return jnp.where(kept, ix * NYG + iy, DUMP)


def _sc_body(x_ref, g_ref, out_ref, grid, featb, gbuf, idx2d,
             idx_tail, sem):
    cid = lax.axis_index("c")
    sid = lax.axis_index("s")
    c0 = cid * CH
    lane = lax.iota(jnp.int32, 16)

    def _geom_vec(i):
        off = i * 48
        fx = plsc.load_gather(gbuf, [lane * 3 + off])
        fy = plsc.load_gather(gbuf, [lane * 3 + (off + 1)])
        fz = plsc.load_gather(gbuf, [lane * 3 + (off + 2)])
        return _voxelize(fx, fy, fz)

    def _do_chunk(base, npts):
        cp = pltpu.async_copy(
            x_ref.at[pl.ds(base, npts), pl.ds(c0, CH)],
            featb.at[pl.ds(0, npts)], sem)
        pltpu.sync_copy(g_ref.at[pl.ds(base * 3, npts * 3)],
                        gbuf.at[pl.ds(0, npts * 3)])
        nvec_full = (npts // 128) * 8

        def _vec(i, _):
            idx2d[i // 8, pl.ds((i % 8) * 16, 16)] = _geom_vec(i)
            return 0
        lax.fori_loop(0, nvec_full, _vec, 0)

        if npts % 128:
            rem = npts - (npts // 128) * 128
            v0 = nvec_full
            idx_tail[pl.ds(0, 16)] = _geom_vec(v0)
            val = _geom_vec(v0 + 1)
            plsc.store_scatter(idx_tail, [lane + 16], val,
                               mask=lane < rem - 16)
        cp.wait()
        for j in range(npts // 128):
            pltpu.sync_copy(featb.at[pl.ds(j * 128, 128)],
                            grid.at[idx2d.at[j]], add=True)
        if npts % 128:
            pltpu.sync_copy(featb.at[pl.ds((npts // 128) * 128, rem)],
                            grid.at[idx_tail], add=True)

    for b in range(B):
        def _zb(i, _):
            featb[i, pl.ds(0, 16)] = jnp.zeros((16,), jnp.float32)
            featb[i, pl.ds(16, 16)] = jnp.zeros((16,), jnp.float32)
            return 0
        lax.fori_loop(0, CHUNK, _zb, 0)
        r0 = sid * 2501
        pltpu.sync_copy(featb, grid.at[pl.ds(r0, CHUNK)])
        pltpu.sync_copy(featb, grid.at[pl.ds(r0 + CHUNK, CHUNK)])
        pltpu.sync_copy(featb.at[pl.ds(0, 2501 - 2 * CHUNK)],
                        grid.at[pl.ds(r0 + 2 * CHUNK, 2501 - 2 * CHUNK)])
        plsc.subcore_barrier()

        bb = b * PPB

        def _chunk_loop(t, _):
            q = t * NTILES + sid

            @pl.when(q < NFULL)
            def _():
                _do_chunk(bb + q * CHUNK, CHUNK)
            return 0
        lax.fori_loop(0, (NFULL + NTILES - 1) // NTILES, _chunk_loop, 0)

        @pl.when(sid == NTILES - 1)
        def _():
            _do_chunk(bb + NFULL * CHUNK, TAIL)
        plsc.subcore_barrier()

        o0 = sid * 2500
        pltpu.sync_copy(grid.at[pl.ds(o0, 2500)],
                        out_ref.at[b, pl.ds(o0, 2500), pl.ds(c0, CH)])
        plsc.subcore_barrier()


_sc_call = pl.kernel(
    _sc_body,
    out_type=jax.ShapeDtypeStruct((B, NXG * NYG, C), jnp.float32),
    mesh=plsc.VectorSubcoreMesh(core_axis_name="c", subcore_axis_name="s"),
    scratch_types=[
        pltpu.VMEM_SHARED((GROWS, CH), jnp.float32),
        pltpu.VMEM((CHUNK, CH), jnp.float32),
        pltpu.VMEM((3 * CHUNK,), jnp.float32),
        pltpu.VMEM((8, 128), jnp.int32),
        pltpu.VMEM((24,), jnp.int32),
        pltpu.SemaphoreType.DMA,
    ],
    compiler_params=pltpu.CompilerParams(use_tc_tiling_on_sc=False,
                                         needs_layout_passes=False),
)


_VB = 4096
_NJ = (NXG * NYG) // _VB
_REM = NXG * NYG - _NJ * _VB


def _tr_body(v_ref, o_ref, tbuf, tbuf2, sems, rsem):
    b = pl.program_id(0)
    j = pl.program_id(1)
    g = b * _NJ + j

    @pl.when(j < _NJ)
    def _():
        slot = lax.rem(g, 2)

        @pl.when(g >= 2)
        def _():
            pltpu.make_async_copy(
                tbuf.at[slot], o_ref.at[b, :, pl.ds(0, _VB)],
                sems.at[slot]).wait()
        tbuf[slot] = v_ref[0].T
        pltpu.make_async_copy(
            tbuf.at[slot], o_ref.at[b, :, pl.ds(j * _VB, _VB)],
            sems.at[slot]).start()

    @pl.when(j == _NJ)
    def _():
        tbuf2[...] = v_ref[0, :_REM, :].T
        cp = pltpu.make_async_copy(
            tbuf2, o_ref.at[b, :, pl.ds(_NJ * _VB, _REM)], rsem)
        cp.start()
        cp.wait()

    @pl.when((b == B - 1) & (j == _NJ))
    def _():
        for s_ in range(2):
            pltpu.make_async_copy(
                tbuf.at[s_], o_ref.at[B - 1, :, pl.ds(0, _VB)],
                sems.at[s_]).wait()


_tr_call = pl.pallas_call(
    _tr_body,
    grid=(B, _NJ + 1),
    in_specs=[pl.BlockSpec((1, _VB, C), lambda b, j: (b, j, 0))],
    out_specs=pl.BlockSpec(memory_space=pltpu.MemorySpace.HBM),
    out_shape=jax.ShapeDtypeStruct((B, C, NXG * NYG), jnp.float32),
    scratch_shapes=[pltpu.VMEM((2, C, _VB), jnp.float32),
                    pltpu.VMEM((C, _REM), jnp.float32),
                    pltpu.SemaphoreType.DMA((2,)),
                    pltpu.SemaphoreType.DMA],
)


def kernel(x, geom_feats):
    xf = x.reshape(NP, C)
    gf = geom_feats.reshape(NP * 3)
    vox = _sc_call(xf, gf)
    out = _tr_call(vox)
    return out.reshape(B, C, NXG, NYG)


# speedup vs baseline: 2.2783x; 1.0676x over previous
import jax
import jax.numpy as jnp
from jax import lax
from jax.experimental import pallas as pl
from jax.experimental.pallas import tpu as pltpu
from jax.experimental.pallas import tpu_sc as plsc

B, N, D, H, W, C = 4, 6, 41, 14, 30, 64
NXG, NYG = 200, 200
NP = B * N * D * H * W
PPB = NP // B
CHUNK = 1024
NFULL = PPB // CHUNK
TAIL = PPB - NFULL * CHUNK
NTILES = 16
GROWS = 40016
DUMP = 40000
CH = 32


def _voxelize(fx, fy, fz):
    ix = ((fx + 50.0) / 0.5).astype(jnp.int32)
    iy = ((fy + 50.0) / 0.5).astype(jnp.int32)
    iz = ((fz + 10.0) / 20.0).astype(jnp.int32)
    kept = ((ix >= 0) & (ix < NXG) & (iy >= 0) & (iy < NYG) & (iz == 0))
    return jnp.where(kept, ix * NYG + iy, DUMP)


def _sc_body(x_ref, g_ref, out_ref, grid, featb, gbuf, idx2d,
             idx_tail, sem):
    cid = lax.axis_index("c")
    sid = lax.axis_index("s")
    c0 = cid * CH
    lane = lax.iota(jnp.int32, 16)

    def _geom_vec(i):
        off = i * 48
        fx = plsc.load_gather(gbuf, [lane * 3 + off])
        fy = plsc.load_gather(gbuf, [lane * 3 + (off + 1)])
        fz = plsc.load_gather(gbuf, [lane * 3 + (off + 2)])
        return _voxelize(fx, fy, fz)

    def _do_chunk(base, npts):
        cp = pltpu.async_copy(
            x_ref.at[pl.ds(base, npts), pl.ds(c0, CH)],
            featb.at[pl.ds(0, npts)], sem)
        pltpu.sync_copy(g_ref.at[pl.ds(base * 3, npts * 3)],
                        gbuf.at[pl.ds(0, npts * 3)])
        nvec_full = (npts // 128) * 8

        def _vec(i, _):
            idx2d[i // 8, pl.ds((i % 8) * 16, 16)] = _geom_vec(i)
            return 0
        lax.fori_loop(0, nvec_full, _vec, 0)

        if npts % 128:
            rem = npts - (npts // 128) * 128
            v0 = nvec_full
            idx_tail[pl.ds(0, 16)] = _geom_vec(v0)
            val = _geom_vec(v0 + 1)
            plsc.store_scatter(idx_tail, [lane + 16], val,
                               mask=lane < rem - 16)
        cp.wait()
        for j in range(npts // 128):
            pltpu.sync_copy(featb.at[pl.ds(j * 128, 128)],
                            grid.at[idx2d.at[j]], add=True)
        if npts % 128:
            pltpu.sync_copy(featb.at[pl.ds((npts // 128) * 128, rem)],
                            grid.at[idx_tail], add=True)

    for b in range(B):
        def _zb(i, _):
            featb[i, pl.ds(0, 16)] = jnp.zeros((16,), jnp.float32)
            featb[i, pl.ds(16, 16)] = jnp.zeros((16,), jnp.float32)
            return 0
        lax.fori_loop(0, CHUNK, _zb, 0)
        r0 = sid * 2501
        pltpu.sync_copy(featb, grid.at[pl.ds(r0, CHUNK)])
        pltpu.sync_copy(featb, grid.at[pl.ds(r0 + CHUNK, CHUNK)])
        pltpu.sync_copy(featb.at[pl.ds(0, 2501 - 2 * CHUNK)],
                        grid.at[pl.ds(r0 + 2 * CHUNK, 2501 - 2 * CHUNK)])
        plsc.subcore_barrier()

        bb = b * PPB

        def _chunk_loop(t, _):
            q = t * NTILES + sid

            @pl.when(q < NFULL)
            def _():
                _do_chunk(bb + q * CHUNK, CHUNK)
            return 0
        lax.fori_loop(0, (NFULL + NTILES - 1) // NTILES, _chunk_loop, 0)

        @pl.when(sid == NTILES - 1)
        def _():
            _do_chunk(bb + NFULL * CHUNK, TAIL)
        plsc.subcore_barrier()

        o0 = sid * 2500
        pltpu.sync_copy(grid.at[pl.ds(o0, 2500)],
                        out_ref.at[b, pl.ds(o0, 2500), pl.ds(c0, CH)])
        plsc.subcore_barrier()


_sc_call = pl.kernel(
    _sc_body,
    out_type=jax.ShapeDtypeStruct((B, NXG * NYG, 128), jnp.float32),
    mesh=plsc.VectorSubcoreMesh(core_axis_name="c", subcore_axis_name="s"),
    scratch_types=[
        pltpu.VMEM_SHARED((GROWS, CH), jnp.float32),
        pltpu.VMEM((CHUNK, CH), jnp.float32),
        pltpu.VMEM((3 * CHUNK,), jnp.float32),
        pltpu.VMEM((8, 128), jnp.int32),
        pltpu.VMEM((24,), jnp.int32),
        pltpu.SemaphoreType.DMA,
    ],
    compiler_params=pltpu.CompilerParams(use_tc_tiling_on_sc=False,
                                         needs_layout_passes=False),
)


_VB = 4096
_NJ = (NXG * NYG) // _VB
_REM = NXG * NYG - _NJ * _VB


def _tr_body(v_ref, o_ref, tbuf, tbuf2, sems, rsem):
    b = pl.program_id(0)
    j = pl.program_id(1)
    g = b * _NJ + j

    @pl.when(j < _NJ)
    def _():
        slot = lax.rem(g, 2)

        @pl.when(g >= 2)
        def _():
            pltpu.make_async_copy(
                tbuf.at[slot], o_ref.at[b, :, pl.ds(0, _VB)],
                sems.at[slot]).wait()
        tbuf[slot] = v_ref[0, :, :C].T
        pltpu.make_async_copy(
            tbuf.at[slot], o_ref.at[b, :, pl.ds(j * _VB, _VB)],
            sems.at[slot]).start()

    @pl.when(j == _NJ)
    def _():
        tbuf2[...] = v_ref[0, :_REM, :C].T
        cp = pltpu.make_async_copy(
            tbuf2, o_ref.at[b, :, pl.ds(_NJ * _VB, _REM)], rsem)
        cp.start()
        cp.wait()

    @pl.when((b == B - 1) & (j == _NJ))
    def _():
        for s_ in range(2):
            pltpu.make_async_copy(
                tbuf.at[s_], o_ref.at[B - 1, :, pl.ds(0, _VB)],
                sems.at[s_]).wait()


_tr_call = pl.pallas_call(
    _tr_body,
    grid=(B, _NJ + 1),
    in_specs=[pl.BlockSpec((1, _VB, 128), lambda b, j: (b, j, 0))],
    out_specs=pl.BlockSpec(memory_space=pltpu.MemorySpace.HBM),
    out_shape=jax.ShapeDtypeStruct((B, C, NXG * NYG), jnp.float32),
    scratch_shapes=[pltpu.VMEM((2, C, _VB), jnp.float32),
                    pltpu.VMEM((C, _REM), jnp.float32),
                    pltpu.SemaphoreType.DMA((2,)),
                    pltpu.SemaphoreType.DMA],
)


def kernel(x, geom_feats):
    xf = x.reshape(NP, C)
    gf = geom_feats.reshape(NP * 3)
    vox = _sc_call(xf, gf)
    out = _tr_call(vox)
    return out.reshape(B, C, NXG, NYG)
